# Initial kernel scaffold; baseline (speedup 1.0000x reference)
import jax
import jax.numpy as jnp
from jax.experimental import pallas as pl


def kernel(x1, x2, W, edge_index1, edge_index2):
    raise NotImplementedError("write your pallas kernel here")


# speedup vs baseline: 3.9187x; 3.9187x over previous
import jax
import jax.numpy as jnp
from jax import lax
from jax.experimental import pallas as pl
from jax.experimental.pallas import tpu as pltpu, tpu_sc as plsc

N = 10000
E = 320000
D = 128
H = 128

CHUNK = 128
NUM_TILES = 16
CHUNKS_PER_TILE = 160
CHUNKS_PER_BRANCH = NUM_TILES * CHUNKS_PER_TILE
E_PAD = CHUNKS_PER_BRANCH * CHUNK - E
DUMMY_DST = N
ROWS_MAIN = 624
TAIL_ROWS = N - NUM_TILES * ROWS_MAIN
N_PAD = 10240
BLK = 32


def _sc_body(x_hbm, src_hbm, dst_hbm, agg_out, cnt_out,
             src_blk, dst_blk, rows, ones, zcnt,
             agg_sh, cnt_sh, gsem):
    c = lax.axis_index("c")
    s = lax.axis_index("s")

    zero16 = jnp.zeros((16,), jnp.float32)
    one16 = jnp.ones((16,), jnp.float32)

    def fill_rows_zero(i, carry):
        for k in range(8):
            rows[i, pl.ds(16 * k, 16)] = zero16
        return carry

    lax.fori_loop(0, CHUNK, fill_rows_zero, 0)

    def fill_zcnt_ones(i, carry):
        zcnt[i, :] = zero16
        ones[i, :] = one16
        return carry

    lax.fori_loop(0, CHUNK, fill_zcnt_ones, 0)

    for i in range(5):
        pltpu.sync_copy(rows, agg_sh.at[pl.ds(s * 640 + i * CHUNK, CHUNK)])
        pltpu.sync_copy(zcnt, cnt_sh.at[pl.ds(s * 640 + i * CHUNK, CHUNK)])

    plsc.subcore_barrier()

    start = (c * NUM_TILES + s) * CHUNKS_PER_TILE

    def idx_block(b, carry):
        pltpu.sync_copy(src_hbm.at[pl.ds(start + b * BLK, BLK)], src_blk)
        pltpu.sync_copy(dst_hbm.at[pl.ds(start + b * BLK, BLK)], dst_blk)

        def edge_chunk(j, carry2):
            pltpu.async_copy(x_hbm.at[src_blk.at[j]], rows, gsem).wait()
            pltpu.sync_copy(rows, agg_sh.at[dst_blk.at[j]], add=True)
            pltpu.sync_copy(ones, cnt_sh.at[dst_blk.at[j]], add=True)
            return carry2

        lax.fori_loop(0, BLK, edge_chunk, 0)
        return carry

    lax.fori_loop(0, CHUNKS_PER_TILE // BLK, idx_block, 0)

    plsc.subcore_barrier()

    base = s * ROWS_MAIN
    pltpu.sync_copy(agg_sh.at[pl.ds(base, ROWS_MAIN)],
                    agg_out.at[pl.ds(c * N + base, ROWS_MAIN)])
    pltpu.sync_copy(cnt_sh.at[pl.ds(base, ROWS_MAIN)],
                    cnt_out.at[pl.ds(c * N + base, ROWS_MAIN)])

    @pl.when(s == NUM_TILES - 1)
    def _tail():
        tbase = NUM_TILES * ROWS_MAIN
        pltpu.sync_copy(agg_sh.at[pl.ds(tbase, TAIL_ROWS)],
                        agg_out.at[pl.ds(c * N + tbase, TAIL_ROWS)])
        pltpu.sync_copy(cnt_sh.at[pl.ds(tbase, TAIL_ROWS)],
                        cnt_out.at[pl.ds(c * N + tbase, TAIL_ROWS)])


@jax.jit
def _sc_aggregate(x_flat, src2d, dst2d):
    mesh = plsc.VectorSubcoreMesh(core_axis_name="c", subcore_axis_name="s")
    return pl.kernel(
        _sc_body,
        out_type=[
            jax.ShapeDtypeStruct((2 * N, D), jnp.float32),
            jax.ShapeDtypeStruct((2 * N, 16), jnp.float32),
        ],
        mesh=mesh,
        compiler_params=pltpu.CompilerParams(use_tc_tiling_on_sc=False),
        scratch_types=[
            pltpu.VMEM((BLK, CHUNK), jnp.int32),
            pltpu.VMEM((BLK, CHUNK), jnp.int32),
            pltpu.VMEM((CHUNK, D), jnp.float32),
            pltpu.VMEM((CHUNK, 16), jnp.float32),
            pltpu.VMEM((CHUNK, 16), jnp.float32),
            pltpu.VMEM_SHARED((N_PAD, D), jnp.float32),
            pltpu.VMEM_SHARED((N_PAD, 16), jnp.float32),
            pltpu.SemaphoreType.DMA,
        ],
    )(x_flat, src2d, dst2d)


def _tc_body(agg_ref, cnt_ref, w_ref, out_ref):
    w = w_ref[...]
    embs = []
    for c in range(2):
        a = agg_ref[c * N:(c + 1) * N, :]
        deg = cnt_ref[c * N:(c + 1) * N, 0:1]
        a = a / jnp.maximum(deg, 1.0)
        h = jnp.maximum(
            jax.lax.dot(a, w, preferred_element_type=jnp.float32), 0.0)
        embs.append(jnp.sum(h, axis=0, keepdims=True) / float(N))
    out_ref[...] = jnp.sum(embs[0] * embs[1]).reshape(1, 1)


@jax.jit
def _tc_finish(agg, cnt, W):
    return pl.pallas_call(
        _tc_body,
        out_shape=jax.ShapeDtypeStruct((1, 1), jnp.float32),
    )(agg, cnt, W)


def kernel(x1, x2, W, edge_index1, edge_index2):
    x_flat = jnp.concatenate([x1, x2], axis=0)
    src_pad = jnp.zeros((E_PAD,), jnp.int32)
    dst_pad = jnp.full((E_PAD,), DUMMY_DST, jnp.int32)
    src2d = jnp.concatenate(
        [edge_index1[0], src_pad, edge_index2[0] + N, src_pad]).reshape(-1, CHUNK)
    dst2d = jnp.concatenate(
        [edge_index1[1], dst_pad, edge_index2[1], dst_pad]).reshape(-1, CHUNK)
    agg, cnt = _sc_aggregate(x_flat, src2d, dst2d)
    out = _tc_finish(agg, cnt, W)
    return out[0, 0]


# speedup vs baseline: 3.9540x; 1.0090x over previous
import jax
import jax.numpy as jnp
from jax import lax
from jax.experimental import pallas as pl
from jax.experimental.pallas import tpu as pltpu, tpu_sc as plsc

N = 10000
E = 320000
D = 128
H = 128

DA = 144
CHUNK = 128
NUM_TILES = 16
CHUNKS_PER_TILE = 160
CHUNKS_PER_BRANCH = NUM_TILES * CHUNKS_PER_TILE
E_PAD = CHUNKS_PER_BRANCH * CHUNK - E
DUMMY_DST = N
ROWS_MAIN = 624
TAIL_ROWS = N - NUM_TILES * ROWS_MAIN
N_PAD = 10112
ZPT = 632
BLK = 8
NBLK = CHUNKS_PER_TILE // BLK


def _sc_body(x_hbm, src_hbm, dst_hbm, agg_out,
             src_blk, dst_blk, rows_a, rows_b,
             agg_sh, gsem_a, gsem_b):
    c = lax.axis_index("c")
    s = lax.axis_index("s")

    zero16 = jnp.zeros((16,), jnp.float32)

    def fill_rows_zero(i, carry):
        for k in range(DA // 16):
            rows_a[i, pl.ds(16 * k, 16)] = zero16
            rows_b[i, pl.ds(16 * k, 16)] = zero16
        return carry

    lax.fori_loop(0, CHUNK, fill_rows_zero, 0)

    for i in range(4):
        pltpu.sync_copy(rows_a, agg_sh.at[pl.ds(s * ZPT + i * CHUNK, CHUNK)])
    pltpu.sync_copy(rows_b.at[pl.ds(0, ZPT - 4 * CHUNK)],
                    agg_sh.at[pl.ds(s * ZPT + 4 * CHUNK, ZPT - 4 * CHUNK)])

    plsc.subcore_barrier()

    tile_start = (c * NUM_TILES + s) * CHUNKS_PER_TILE

    def gather(buf, idx_row, sem):
        return pltpu.async_copy(x_hbm.at[src_blk.at[idx_row]], buf, sem)

    def gather_wait(buf, idx_row, sem):
        pltpu.make_async_copy(x_hbm.at[src_blk.at[idx_row]], buf, sem).wait()

    def scatter(buf, idx_row):
        pltpu.sync_copy(buf, agg_sh.at[dst_blk.at[idx_row]], add=True)

    pltpu.sync_copy(src_hbm.at[pl.ds(tile_start, BLK)], src_blk)
    pltpu.sync_copy(dst_hbm.at[pl.ds(tile_start, BLK)], dst_blk)
    gather(rows_a, 0, gsem_a)

    def block_body(b, carry):
        for p in range(BLK // 2):
            j0, j1 = 2 * p, 2 * p + 1
            gather_wait(rows_a, j0, gsem_a)
            gather(rows_b, j1, gsem_b)
            scatter(rows_a, j0)
            gather_wait(rows_b, j1, gsem_b)
            if p < BLK // 2 - 1:
                gather(rows_a, j0 + 2, gsem_a)
                scatter(rows_b, j1)
            else:
                @pl.when(b < NBLK - 1)
                def _refill():
                    pltpu.sync_copy(
                        src_hbm.at[pl.ds(tile_start + (b + 1) * BLK, BLK)],
                        src_blk)
                    gather(rows_a, 0, gsem_a)

                scatter(rows_b, j1)

        @pl.when(b < NBLK - 1)
        def _restage_dst():
            pltpu.sync_copy(
                dst_hbm.at[pl.ds(tile_start + (b + 1) * BLK, BLK)], dst_blk)

        return carry

    lax.fori_loop(0, NBLK, block_body, 0)

    plsc.subcore_barrier()

    base = s * ROWS_MAIN
    pltpu.sync_copy(agg_sh.at[pl.ds(base, ROWS_MAIN)],
                    agg_out.at[pl.ds(c * N + base, ROWS_MAIN)])

    @pl.when(s == NUM_TILES - 1)
    def _tail():
        tbase = NUM_TILES * ROWS_MAIN
        pltpu.sync_copy(agg_sh.at[pl.ds(tbase, TAIL_ROWS)],
                        agg_out.at[pl.ds(c * N + tbase, TAIL_ROWS)])


@jax.jit
def _sc_aggregate(x_aug, src2d, dst2d):
    mesh = plsc.VectorSubcoreMesh(core_axis_name="c", subcore_axis_name="s")
    return pl.kernel(
        _sc_body,
        out_type=jax.ShapeDtypeStruct((2 * N, DA), jnp.float32),
        mesh=mesh,
        compiler_params=pltpu.CompilerParams(use_tc_tiling_on_sc=False),
        scratch_types=[
            pltpu.VMEM((BLK, CHUNK), jnp.int32),
            pltpu.VMEM((BLK, CHUNK), jnp.int32),
            pltpu.VMEM((CHUNK, DA), jnp.float32),
            pltpu.VMEM((CHUNK, DA), jnp.float32),
            pltpu.VMEM_SHARED((N_PAD, DA), jnp.float32),
            pltpu.SemaphoreType.DMA,
            pltpu.SemaphoreType.DMA,
        ],
    )(x_aug, src2d, dst2d)


def _tc_body(agg_ref, w_ref, out_ref):
    w = w_ref[...]
    embs = []
    for c in range(2):
        a = agg_ref[c * N:(c + 1) * N, 0:D]
        deg = agg_ref[c * N:(c + 1) * N, D:D + 1]
        a = a / jnp.maximum(deg, 1.0)
        h = jnp.maximum(
            jax.lax.dot(a, w, preferred_element_type=jnp.float32), 0.0)
        embs.append(jnp.sum(h, axis=0, keepdims=True) / float(N))
    out_ref[...] = jnp.sum(embs[0] * embs[1]).reshape(1, 1)


@jax.jit
def _tc_finish(agg, W):
    return pl.pallas_call(
        _tc_body,
        out_shape=jax.ShapeDtypeStruct((1, 1), jnp.float32),
    )(agg, W)


def kernel(x1, x2, W, edge_index1, edge_index2):
    x_aug = jnp.concatenate(
        [jnp.concatenate([x1, x2], axis=0),
         jnp.ones((2 * N, DA - D), jnp.float32)], axis=1)
    src_pad = jnp.zeros((E_PAD,), jnp.int32)
    dst_pad = jnp.full((E_PAD,), DUMMY_DST, jnp.int32)
    src2d = jnp.concatenate(
        [edge_index1[0], src_pad, edge_index2[0] + N, src_pad]).reshape(-1, CHUNK)
    dst2d = jnp.concatenate(
        [edge_index1[1], dst_pad, edge_index2[1], dst_pad]).reshape(-1, CHUNK)
    agg = _sc_aggregate(x_aug, src2d, dst2d)
    out = _tc_finish(agg, W)
    return out[0, 0]
